# Initial kernel scaffold; baseline (speedup 1.0000x reference)
import functools

import jax
import jax.numpy as jnp
from jax import lax
from jax.experimental import pallas as pl
from jax.experimental.pallas import tpu as pltpu
from jax.experimental.pallas import tpu_sc as plsc

LANES = 16
CHUNK = 128
EPS = 1e-12

_info = plsc.get_sparse_core_info()
_NC, _NS = _info.num_cores, _info.num_subcores
NW = _NC * _NS


def _rsqrt(x):
    i = lax.bitcast_convert_type(x, jnp.int32)
    i = jnp.int32(0x5F3759DF) - (i >> 1)
    y = lax.bitcast_convert_type(i, jnp.float32)
    for _ in range(3):
        y = y * (1.5 - 0.5 * x * y * y)
    return y


def _make_sc_kernel(N, S, H):
    NH = H // LANES
    per_w = N // NW
    nchunks = per_w // CHUNK
    inv_h = 1.0 / H

    def body(ids_hbm, tt_hbm, word_hbm, pos_hbm, consts_hbm, out_hbm,
             ids_v, tt_v, rows_v, pos_v, cv, sem):
        wid = lax.axis_index("s") * _NC + lax.axis_index("c")
        pltpu.sync_copy(pos_hbm, pos_v)
        pltpu.sync_copy(consts_hbm, cv)

        def chunk_body(c, carry):
            base = wid * per_w + c * CHUNK
            pltpu.sync_copy(ids_hbm.at[pl.ds(base, CHUNK)], ids_v)
            pltpu.sync_copy(tt_hbm.at[pl.ds(base, CHUNK)], tt_v)
            pltpu.async_copy(word_hbm.at[ids_v], rows_v, sem).wait()
            p0 = base % S

            def tok_body(t, tc):
                p = p0 + t
                p = jnp.where(p >= S, p - S, p)
                ttb = plsc.load_gather(tt_v, [jnp.full((LANES,), t, jnp.int32)])
                ttf = ttb.astype(jnp.float32)
                xs = []
                s = None
                q = None
                for j in range(NH):
                    sl = pl.ds(j * LANES, LANES)
                    x = rows_v[t, sl] + pos_v[p, sl] + cv[0, sl] + ttf * cv[1, sl]
                    xs.append(x)
                    s = x if s is None else s + x
                    q = x * x if q is None else q + x * x
                m = jnp.sum(s) * inv_h
                var = jnp.maximum(jnp.sum(q) * inv_h - m * m, 0.0)
                r = _rsqrt(var + EPS)
                for j in range(NH):
                    sl = pl.ds(j * LANES, LANES)
                    rows_v[t, sl] = (xs[j] - m) * r * cv[2, sl] + cv[3, sl]
                return tc

            lax.fori_loop(0, CHUNK, tok_body, 0)
            pltpu.sync_copy(rows_v, out_hbm.at[pl.ds(base, CHUNK)])
            return carry

        lax.fori_loop(0, nchunks, chunk_body, 0)

    return pl.kernel(
        body,
        out_type=jax.ShapeDtypeStruct((N, H), jnp.float32),
        mesh=plsc.VectorSubcoreMesh(core_axis_name="c", subcore_axis_name="s"),
        scratch_types=[
            pltpu.VMEM((CHUNK,), jnp.int32),
            pltpu.VMEM((CHUNK,), jnp.int32),
            pltpu.VMEM((CHUNK, H), jnp.float32),
            pltpu.VMEM((S, H), jnp.float32),
            pltpu.VMEM((4, H), jnp.float32),
            pltpu.SemaphoreType.DMA,
        ],
    )


def kernel(input_ids, token_type_ids, word_table, pos_table, type_table,
           ln_gamma, ln_beta):
    B, S = input_ids.shape
    H = word_table.shape[1]
    N = B * S
    ids = input_ids.reshape(N).astype(jnp.int32)
    tt = token_type_ids.reshape(N).astype(jnp.int32)
    pos = pos_table[:S]
    consts = jnp.stack(
        [type_table[0], type_table[1] - type_table[0], ln_gamma, ln_beta])
    out = _make_sc_kernel(N, S, H)(ids, tt, word_table, pos, consts)
    return out.reshape(B, S, H)


# speedup vs baseline: 1.6054x; 1.6054x over previous
import functools

import jax
import jax.numpy as jnp
from jax import lax
from jax.experimental import pallas as pl
from jax.experimental.pallas import tpu as pltpu
from jax.experimental.pallas import tpu_sc as plsc

LANES = 16
CHUNK = 128
EPS = 1e-12

_info = plsc.get_sparse_core_info()
_NC, _NS = _info.num_cores, _info.num_subcores
NW = _NC * _NS


def _rsqrt(x):
    i = lax.bitcast_convert_type(x, jnp.int32)
    i = jnp.int32(0x5F3759DF) - (i >> 1)
    y = lax.bitcast_convert_type(i, jnp.float32)
    for _ in range(3):
        y = y * (1.5 - 0.5 * x * y * y)
    return y


def _make_sc_kernel(N, S, H):
    NH = H // LANES
    per_w = N // NW
    nchunks = per_w // CHUNK
    inv_h = 1.0 / H

    def body(ids_hbm, tt_hbm, word_hbm, pos_hbm, consts_hbm, out_hbm,
             ids_v, tt_v, rows_v, pos_v, cv, sem):
        wid = lax.axis_index("s") * _NC + lax.axis_index("c")
        pltpu.sync_copy(pos_hbm, pos_v)
        pltpu.sync_copy(consts_hbm, cv)

        iota = lax.iota(jnp.int32, LANES)
        lane_idx = [jnp.full((LANES, 1), k, jnp.int32) for k in range(LANES)]
        xor_idx = {d: lax.reshape(iota ^ d, (LANES, 1)) for d in (8, 4, 2, 1)}
        gd = lax.GatherDimensionNumbers(
            offset_dims=(), collapsed_slice_dims=(0,), start_index_map=(0,))

        def perm(vec, idx):
            return lax.gather(vec, idx, gd, slice_sizes=(1,),
                              mode=lax.GatherScatterMode.PROMISE_IN_BOUNDS)

        def bcast_lane(vec, k):
            return perm(vec, lane_idx[k])

        def lane_sum(vec):
            for d in (8, 4, 2, 1):
                vec = vec + perm(vec, xor_idx[d])
            return vec

        def chunk_body(c, carry):
            base = wid * per_w + c * CHUNK
            pltpu.sync_copy(ids_hbm.at[pl.ds(base, CHUNK)], ids_v)
            pltpu.sync_copy(tt_hbm.at[pl.ds(base, CHUNK)], tt_v)
            pltpu.async_copy(word_hbm.at[ids_v], rows_v, sem).wait()
            p0 = base % S

            def grp_body(g, tc):
                tb = g * LANES
                tg = tt_v[pl.ds(tb, LANES)]
                for k in range(LANES):
                    t = tb + k
                    p = p0 + t
                    p = jnp.where(p >= S, p - S, p)
                    ttf = bcast_lane(tg, k).astype(jnp.float32)
                    xs = []
                    s = None
                    q = None
                    for j in range(NH):
                        sl = pl.ds(j * LANES, LANES)
                        x = (rows_v[t, sl] + pos_v[p, sl] + cv[0, sl]
                             + ttf * cv[1, sl])
                        xs.append(x)
                        s = x if s is None else s + x
                        q = x * x if q is None else q + x * x
                    m = lane_sum(s) * inv_h
                    var = jnp.maximum(lane_sum(q) * inv_h - m * m, 0.0)
                    r = _rsqrt(var + EPS)
                    for j in range(NH):
                        sl = pl.ds(j * LANES, LANES)
                        rows_v[t, sl] = (xs[j] - m) * r * cv[2, sl] + cv[3, sl]
                return tc

            lax.fori_loop(0, CHUNK // LANES, grp_body, 0)
            pltpu.sync_copy(rows_v, out_hbm.at[pl.ds(base, CHUNK)])
            return carry

        lax.fori_loop(0, nchunks, chunk_body, 0)

    return pl.kernel(
        body,
        out_type=jax.ShapeDtypeStruct((N, H), jnp.float32),
        mesh=plsc.VectorSubcoreMesh(core_axis_name="c", subcore_axis_name="s"),
        scratch_types=[
            pltpu.VMEM((CHUNK,), jnp.int32),
            pltpu.VMEM((CHUNK,), jnp.int32),
            pltpu.VMEM((CHUNK, H), jnp.float32),
            pltpu.VMEM((S, H), jnp.float32),
            pltpu.VMEM((4, H), jnp.float32),
            pltpu.SemaphoreType.DMA,
        ],
    )


def kernel(input_ids, token_type_ids, word_table, pos_table, type_table,
           ln_gamma, ln_beta):
    B, S = input_ids.shape
    H = word_table.shape[1]
    N = B * S
    ids = input_ids.reshape(N).astype(jnp.int32)
    tt = token_type_ids.reshape(N).astype(jnp.int32)
    pos = pos_table[:S]
    consts = jnp.stack(
        [type_table[0], type_table[1] - type_table[0], ln_gamma, ln_beta])
    out = _make_sc_kernel(N, S, H)(ids, tt, word_table, pos, consts)
    return out.reshape(B, S, H)
